# Initial kernel scaffold; baseline (speedup 1.0000x reference)
import functools

import jax
import jax.numpy as jnp
from jax import lax
from jax.experimental import pallas as pl
from jax.experimental.pallas import tpu as pltpu
from jax.experimental.pallas import tpu_sc as plsc

N_FIELDS = 26
VOCAB = 100000
EMBED_DIM = 32
BATCH = 1024
HIST = 20

ROWS = N_FIELDS * BATCH * HIST
ROWS_PER_FIELD = BATCH * HIST
NUM_WORKERS = 32
ROWS_PER_WORKER = ROWS // NUM_WORKERS
GRP = 128
NGRP = ROWS_PER_WORKER // GRP
LANES = 16

_mesh = plsc.VectorSubcoreMesh(core_axis_name="c", subcore_axis_name="s")


@functools.partial(
    pl.kernel,
    out_type=jax.ShapeDtypeStruct((ROWS, EMBED_DIM), jnp.float32),
    mesh=_mesh,
    scratch_types=[
        pltpu.VMEM((GRP,), jnp.int32),
        pltpu.VMEM((GRP, EMBED_DIM), jnp.float32),
        pltpu.SemaphoreType.DMA,
    ],
)
def _gather_all(idx_hbm, tab_hbm, out_hbm, idx_v, rows_v, sem):
    wid = lax.axis_index("s") * 2 + lax.axis_index("c")
    base_w = wid * ROWS_PER_WORKER

    def body(g, carry):
        base = base_w + g * GRP
        pltpu.sync_copy(idx_hbm.at[pl.ds(base, GRP)], idx_v)
        off = (base // ROWS_PER_FIELD) * VOCAB
        for s in range(GRP // LANES):
            sl = pl.ds(s * LANES, LANES)
            idx_v[sl] = idx_v[sl] + off
        pltpu.async_copy(tab_hbm.at[idx_v], rows_v, sem).wait()
        pltpu.sync_copy(rows_v, out_hbm.at[pl.ds(base, GRP)])
        return carry

    lax.fori_loop(0, NGRP, body, 0)


def kernel(inputs, tables):
    idx = inputs.reshape(ROWS)
    tab = tables.reshape(N_FIELDS * VOCAB, EMBED_DIM)
    out = _gather_all(idx, tab)
    return out.reshape(N_FIELDS, BATCH, HIST, EMBED_DIM)


# speedup vs baseline: 1.0680x; 1.0680x over previous
import functools

import jax
import jax.numpy as jnp
from jax import lax
from jax.experimental import pallas as pl
from jax.experimental.pallas import tpu as pltpu
from jax.experimental.pallas import tpu_sc as plsc

N_FIELDS = 26
VOCAB = 100000
EMBED_DIM = 32
BATCH = 1024
HIST = 20

ROWS = N_FIELDS * BATCH * HIST
ROWS_PER_FIELD = BATCH * HIST
NUM_WORKERS = 32
ROWS_PER_WORKER = ROWS // NUM_WORKERS
GRP = 128
NGRP = ROWS_PER_WORKER // GRP
LANES = 16

_mesh = plsc.VectorSubcoreMesh(core_axis_name="c", subcore_axis_name="s")


@functools.partial(
    pl.kernel,
    out_type=jax.ShapeDtypeStruct((ROWS, EMBED_DIM), jnp.float32),
    mesh=_mesh,
    compiler_params=pltpu.CompilerParams(use_tc_tiling_on_sc=False),
    scratch_types=[
        pltpu.VMEM((GRP,), jnp.int32),
        pltpu.VMEM((GRP, EMBED_DIM), jnp.float32),
        pltpu.SemaphoreType.DMA,
    ],
)
def _gather_all(idx_hbm, tab_hbm, out_hbm, idx_v, rows_v, sem):
    wid = lax.axis_index("s") * 2 + lax.axis_index("c")
    base_w = wid * ROWS_PER_WORKER

    def body(g, carry):
        base = base_w + g * GRP
        pltpu.sync_copy(idx_hbm.at[pl.ds(base, GRP)], idx_v)
        off = (base // ROWS_PER_FIELD) * VOCAB
        for s in range(GRP // LANES):
            sl = pl.ds(s * LANES, LANES)
            idx_v[sl] = idx_v[sl] + off
        pltpu.async_copy(tab_hbm.at[idx_v], rows_v, sem).wait()
        pltpu.sync_copy(rows_v, out_hbm.at[pl.ds(base, GRP)])
        return carry

    lax.fori_loop(0, NGRP, body, 0)


def kernel(inputs, tables):
    idx = inputs.reshape(ROWS)
    tab = tables.reshape(N_FIELDS * VOCAB, EMBED_DIM)
    out = _gather_all(idx, tab)
    return out.reshape(N_FIELDS, BATCH, HIST, EMBED_DIM)


# speedup vs baseline: 1.1672x; 1.0929x over previous
import functools

import jax
import jax.numpy as jnp
from jax import lax
from jax.experimental import pallas as pl
from jax.experimental.pallas import tpu as pltpu
from jax.experimental.pallas import tpu_sc as plsc

N_FIELDS = 26
VOCAB = 100000
EMBED_DIM = 32
BATCH = 1024
HIST = 20

ROWS = N_FIELDS * BATCH * HIST
ROWS_PER_FIELD = BATCH * HIST
NUM_WORKERS = 32
LANES = 16

GRP = 128
NGRP_W = ROWS // (NUM_WORKERS * GRP)
GPB = 5
NBUF = 2
NBI = NGRP_W // GPB
T_OUTER = NBI // NBUF
BUF_ROWS = GPB * GRP

_mesh = plsc.VectorSubcoreMesh(core_axis_name="c", subcore_axis_name="s")


@functools.partial(
    pl.kernel,
    out_type=jax.ShapeDtypeStruct((ROWS, EMBED_DIM), jnp.float32),
    mesh=_mesh,
    compiler_params=pltpu.CompilerParams(use_tc_tiling_on_sc=False),
    scratch_types=[
        pltpu.VMEM((NGRP_W, GRP), jnp.int32),
        pltpu.VMEM((BUF_ROWS, EMBED_DIM), jnp.float32),
        pltpu.VMEM((BUF_ROWS, EMBED_DIM), jnp.float32),
        pltpu.SemaphoreType.DMA,
        pltpu.SemaphoreType.DMA,
        pltpu.SemaphoreType.DMA,
        pltpu.SemaphoreType.DMA,
    ],
)
def _gather_all(idx_hbm, tab_hbm, out_hbm, idx_all, buf0, buf1,
                gsem0, gsem1, ssem0, ssem1):
    bufs = (buf0, buf1)
    gsems = (gsem0, gsem1)
    ssems = (ssem0, ssem1)

    wid = lax.axis_index("s") * 2 + lax.axis_index("c")
    gbase = wid * NGRP_W
    rbase = gbase * GRP

    pltpu.sync_copy(idx_hbm.at[pl.ds(gbase, NGRP_W)], idx_all)

    def add_off(g, carry):
        off = ((rbase + g * GRP) // ROWS_PER_FIELD) * VOCAB
        for s in range(GRP // LANES):
            sl = pl.ds(s * LANES, LANES)
            idx_all[g, sl] = idx_all[g, sl] + off
        return carry

    lax.fori_loop(0, NGRP_W, add_off, 0)

    def fire_gathers(bi, b):
        for j in range(GPB):
            pltpu.async_copy(
                tab_hbm.at[idx_all.at[bi * GPB + j]],
                bufs[b].at[pl.ds(j * GRP, GRP)],
                gsems[b],
            )

    def drain_gathers(b):
        pltpu.make_async_copy(
            out_hbm.at[pl.ds(0, BUF_ROWS)], bufs[b], gsems[b]
        ).wait()

    for b in range(NBUF):
        fire_gathers(b, b)

    def outer(t, carry):
        for b in range(NBUF):
            bi = t * NBUF + b
            drain_gathers(b)
            st = pltpu.async_copy(
                bufs[b],
                out_hbm.at[pl.ds(rbase + bi * BUF_ROWS, BUF_ROWS)],
                ssems[b],
            )
            st.wait()

            @pl.when(t < T_OUTER - 1)
            def _():
                fire_gathers(bi + NBUF, b)

        return carry

    lax.fori_loop(0, T_OUTER, outer, 0)


def kernel(inputs, tables):
    idx = inputs.reshape(ROWS // GRP, GRP)
    tab = tables.reshape(N_FIELDS * VOCAB, EMBED_DIM)
    out = _gather_all(idx, tab)
    return out.reshape(N_FIELDS, BATCH, HIST, EMBED_DIM)


# speedup vs baseline: 3.3158x; 2.8409x over previous
import functools

import jax
import jax.numpy as jnp
from jax import lax
from jax.experimental import pallas as pl
from jax.experimental.pallas import tpu as pltpu
from jax.experimental.pallas import tpu_sc as plsc

N_FIELDS = 26
VOCAB = 100000
EMBED_DIM = 32
BATCH = 1024
HIST = 20
LANES = 16
NUM_WORKERS = 32

_mesh = plsc.VectorSubcoreMesh(core_axis_name="c", subcore_axis_name="s")


@functools.partial(
    pl.kernel,
    out_type=jax.ShapeDtypeStruct((N_FIELDS, HIST, EMBED_DIM, BATCH), jnp.float32),
    mesh=_mesh,
    compiler_params=pltpu.CompilerParams(needs_layout_passes=False),
    scratch_types=[
        pltpu.VMEM((VOCAB,), jnp.float32),
        pltpu.VMEM((HIST, BATCH), jnp.int32),
        pltpu.VMEM((BATCH,), jnp.float32),
        pltpu.SemaphoreType.DMA,
    ],
)
def _lookup_t(idx_hbm, tab_hbm, out_hbm, row_v, idx_v, stage_v, sem):
    w = lax.axis_index("s") * 2 + lax.axis_index("c")

    def per_field(f, carry):
        pltpu.sync_copy(tab_hbm.at[f, w], row_v)
        pltpu.sync_copy(idx_hbm.at[f], idx_v)

        def per_hist(h, carry_h):
            def per_vec(s, carry_s):
                sl = pl.ds(s * LANES, LANES)
                ii = idx_v[h, sl]
                stage_v[sl] = plsc.load_gather(row_v, [ii])
                return carry_s

            lax.fori_loop(0, BATCH // LANES, per_vec, 0)
            pltpu.sync_copy(stage_v, out_hbm.at[f, h, w])
            return carry_h

        lax.fori_loop(0, HIST, per_hist, 0)
        return carry

    lax.fori_loop(0, N_FIELDS, per_field, 0)


def kernel(inputs, tables):
    tab_t = jnp.transpose(tables, (0, 2, 1))
    idx_t = jnp.transpose(inputs, (0, 2, 1))
    out_t = _lookup_t(idx_t, tab_t)
    return jnp.transpose(out_t, (0, 3, 1, 2))


# speedup vs baseline: 3.8214x; 1.1525x over previous
import functools

import jax
import jax.numpy as jnp
from jax import lax
from jax.experimental import pallas as pl
from jax.experimental.pallas import tpu as pltpu
from jax.experimental.pallas import tpu_sc as plsc

N_FIELDS = 26
VOCAB = 100000
EMBED_DIM = 32
BATCH = 1024
HIST = 20
LANES = 16
NVEC = BATCH // LANES

_mesh = plsc.VectorSubcoreMesh(core_axis_name="c", subcore_axis_name="s")


@functools.partial(
    pl.kernel,
    out_type=jax.ShapeDtypeStruct((N_FIELDS, HIST, EMBED_DIM, BATCH), jnp.float32),
    mesh=_mesh,
    compiler_params=pltpu.CompilerParams(needs_layout_passes=False),
    scratch_types=[
        pltpu.VMEM((VOCAB,), jnp.float32),
        pltpu.VMEM((BATCH,), jnp.int32),
        pltpu.VMEM((BATCH,), jnp.int32),
        pltpu.VMEM((BATCH,), jnp.float32),
        pltpu.VMEM((BATCH,), jnp.float32),
        pltpu.SemaphoreType.DMA,
        pltpu.SemaphoreType.DMA,
        pltpu.SemaphoreType.DMA,
        pltpu.SemaphoreType.DMA,
    ],
)
def _lookup_t(idx_hbm, tab_hbm, out_hbm, row_v, idx_a, idx_b, st_a, st_b,
              sem_ia, sem_ib, sem_sa, sem_sb):
    w = lax.axis_index("s") * 2 + lax.axis_index("c")

    def drain(buf, sem):
        if buf.dtype == jnp.int32:
            src = idx_hbm.at[0, 0]
        else:
            src = tab_hbm.at[0, 0, pl.ds(0, BATCH)]
        pltpu.make_async_copy(src, buf, sem).wait()

    def gather_run(idx_buf, st_buf):
        for s in range(NVEC):
            sl = pl.ds(s * LANES, LANES)
            st_buf[sl] = plsc.load_gather(row_v, [idx_buf[sl]])

    pltpu.async_copy(idx_hbm.at[0, 0], idx_a, sem_ia)

    def per_field(f, carry):
        pltpu.sync_copy(tab_hbm.at[f, w], row_v)

        def per_pair(k, carry_k):
            h0 = 2 * k
            step = f * (HIST // 2) + k
            nf = jnp.minimum(lax.select(k == HIST // 2 - 1, f + 1, f),
                             N_FIELDS - 1)
            nh = lax.select(k == HIST // 2 - 1, 0, h0 + 2)

            drain(idx_a, sem_ia)
            pltpu.async_copy(idx_hbm.at[f, h0 + 1], idx_b, sem_ib)

            @pl.when(step > 0)
            def _():
                drain(st_a, sem_sa)
            gather_run(idx_a, st_a)
            pltpu.async_copy(st_a, out_hbm.at[f, h0, w], sem_sa)

            drain(idx_b, sem_ib)
            pltpu.async_copy(idx_hbm.at[nf, nh], idx_a, sem_ia)

            @pl.when(step > 0)
            def _():
                drain(st_b, sem_sb)
            gather_run(idx_b, st_b)
            pltpu.async_copy(st_b, out_hbm.at[f, h0 + 1, w], sem_sb)
            return carry_k

        lax.fori_loop(0, HIST // 2, per_pair, 0)
        return carry

    lax.fori_loop(0, N_FIELDS, per_field, 0)
    drain(idx_a, sem_ia)
    drain(st_a, sem_sa)
    drain(st_b, sem_sb)


def kernel(inputs, tables):
    tab_t = jnp.transpose(tables, (0, 2, 1))
    idx_t = jnp.transpose(inputs, (0, 2, 1))
    out_t = _lookup_t(idx_t, tab_t)
    return jnp.transpose(out_t, (0, 3, 1, 2))


# speedup vs baseline: 3.8623x; 1.0107x over previous
import functools

import jax
import jax.numpy as jnp
from jax import lax
from jax.experimental import pallas as pl
from jax.experimental.pallas import tpu as pltpu
from jax.experimental.pallas import tpu_sc as plsc

N_FIELDS = 26
VOCAB = 100000
EMBED_DIM = 32
BATCH = 1024
HIST = 20
LANES = 16
NVEC = BATCH // LANES

_mesh = plsc.VectorSubcoreMesh(core_axis_name="c", subcore_axis_name="s")


@functools.partial(
    pl.kernel,
    out_type=jax.ShapeDtypeStruct((N_FIELDS, HIST, EMBED_DIM, BATCH), jnp.float32),
    mesh=_mesh,
    compiler_params=pltpu.CompilerParams(needs_layout_passes=False),
    scratch_types=[
        pltpu.VMEM((VOCAB,), jnp.float32),
        pltpu.VMEM((BATCH,), jnp.int32),
        pltpu.VMEM((BATCH,), jnp.int32),
        pltpu.VMEM((BATCH,), jnp.float32),
        pltpu.VMEM((BATCH,), jnp.float32),
        pltpu.SemaphoreType.DMA,
        pltpu.SemaphoreType.DMA,
        pltpu.SemaphoreType.DMA,
        pltpu.SemaphoreType.DMA,
    ],
)
def _lookup_t(idx_hbm, tab_hbm, out_hbm, row_v, idx_a, idx_b, st_a, st_b,
              sem_ia, sem_ib, sem_sa, sem_sb):
    w = lax.axis_index("s") * 2 + lax.axis_index("c")

    def drain(buf, sem):
        if buf.dtype == jnp.int32:
            src = idx_hbm.at[0, 0]
        else:
            src = tab_hbm.at[0, 0, pl.ds(0, BATCH)]
        pltpu.make_async_copy(src, buf, sem).wait()

    def gather_run(idx_buf, st_buf):
        @plsc.parallel_loop(0, NVEC, unroll=8)
        def _body(s):
            sl = pl.ds(s * LANES, LANES)
            st_buf[sl] = plsc.load_gather(row_v, [idx_buf[sl]])

    pltpu.async_copy(idx_hbm.at[0, 0], idx_a, sem_ia)

    def per_field(f, carry):
        pltpu.sync_copy(tab_hbm.at[f, w], row_v)

        def per_pair(k, carry_k):
            h0 = 2 * k
            step = f * (HIST // 2) + k
            nf = jnp.minimum(lax.select(k == HIST // 2 - 1, f + 1, f),
                             N_FIELDS - 1)
            nh = lax.select(k == HIST // 2 - 1, 0, h0 + 2)

            drain(idx_a, sem_ia)
            pltpu.async_copy(idx_hbm.at[f, h0 + 1], idx_b, sem_ib)

            @pl.when(step > 0)
            def _():
                drain(st_a, sem_sa)
            gather_run(idx_a, st_a)
            pltpu.async_copy(st_a, out_hbm.at[f, h0, w], sem_sa)

            drain(idx_b, sem_ib)
            pltpu.async_copy(idx_hbm.at[nf, nh], idx_a, sem_ia)

            @pl.when(step > 0)
            def _():
                drain(st_b, sem_sb)
            gather_run(idx_b, st_b)
            pltpu.async_copy(st_b, out_hbm.at[f, h0 + 1, w], sem_sb)
            return carry_k

        lax.fori_loop(0, HIST // 2, per_pair, 0)
        return carry

    lax.fori_loop(0, N_FIELDS, per_field, 0)
    drain(idx_a, sem_ia)
    drain(st_a, sem_sa)
    drain(st_b, sem_sb)


def kernel(inputs, tables):
    tab_t = jnp.transpose(tables, (0, 2, 1))
    idx_t = jnp.transpose(inputs, (0, 2, 1))
    out_t = _lookup_t(idx_t, tab_t)
    return jnp.transpose(out_t, (0, 3, 1, 2))


# speedup vs baseline: 6.8864x; 1.7830x over previous
import functools

import jax
import jax.numpy as jnp
from jax import lax
from jax.experimental import pallas as pl
from jax.experimental.pallas import tpu as pltpu
from jax.experimental.pallas import tpu_sc as plsc

N_FIELDS = 26
VOCAB = 100000
EMBED_DIM = 32
BATCH = 1024
HIST = 20
LANES = 16
NVEC = BATCH // LANES
NIDX = 4
NST = 8

_mesh = plsc.VectorSubcoreMesh(core_axis_name="c", subcore_axis_name="s")


@functools.partial(
    pl.kernel,
    out_type=jax.ShapeDtypeStruct((N_FIELDS, HIST, EMBED_DIM, BATCH), jnp.float32),
    mesh=_mesh,
    compiler_params=pltpu.CompilerParams(needs_layout_passes=False),
    scratch_types=(
        [pltpu.VMEM((VOCAB,), jnp.float32)]
        + [pltpu.VMEM((BATCH,), jnp.int32) for _ in range(NIDX)]
        + [pltpu.VMEM((BATCH,), jnp.float32) for _ in range(NST)]
        + [pltpu.SemaphoreType.DMA for _ in range(NIDX + NST)]
    ),
)
def _lookup_t(idx_hbm, tab_hbm, out_hbm, *scr):
    row_v = scr[0]
    idx_bufs = scr[1:1 + NIDX]
    st_bufs = scr[1 + NIDX:1 + NIDX + NST]
    idx_sems = scr[1 + NIDX + NST:1 + 2 * NIDX + NST]
    st_sems = scr[1 + 2 * NIDX + NST:]

    w = lax.axis_index("s") * 2 + lax.axis_index("c")

    def drain(buf, sem):
        if buf.dtype == jnp.int32:
            src = idx_hbm.at[0, 0]
        else:
            src = tab_hbm.at[0, 0, pl.ds(0, BATCH)]
        pltpu.make_async_copy(src, buf, sem).wait()

    def gather_run(idx_buf, st_buf):
        @plsc.parallel_loop(0, NVEC, unroll=8)
        def _body(s):
            sl = pl.ds(s * LANES, LANES)
            st_buf[sl] = plsc.load_gather(row_v, [idx_buf[sl]])

    for j in range(NIDX):
        pltpu.async_copy(idx_hbm.at[0, j], idx_bufs[j], idx_sems[j])

    def per_field(f, carry):
        pltpu.sync_copy(tab_hbm.at[f, w], row_v)
        for h in range(HIST):
            jb = h % NIDX
            sb = h % NST
            drain(idx_bufs[jb], idx_sems[jb])
            if h < NST:
                @pl.when(f > 0)
                def _():
                    drain(st_bufs[sb], st_sems[sb])
            else:
                drain(st_bufs[sb], st_sems[sb])
            gather_run(idx_bufs[jb], st_bufs[sb])
            pltpu.async_copy(st_bufs[sb], out_hbm.at[f, h, w], st_sems[sb])
            nh = h + NIDX
            if nh < HIST:
                pltpu.async_copy(idx_hbm.at[f, nh], idx_bufs[jb], idx_sems[jb])
            else:
                nf = jnp.minimum(f + 1, N_FIELDS - 1)
                pltpu.async_copy(idx_hbm.at[nf, nh - HIST], idx_bufs[jb],
                                 idx_sems[jb])
        return carry

    lax.fori_loop(0, N_FIELDS, per_field, 0)
    for j in range(NIDX):
        drain(idx_bufs[j], idx_sems[j])
    for s in range(NST):
        drain(st_bufs[s], st_sems[s])


def kernel(inputs, tables):
    tab_t = jnp.transpose(tables, (0, 2, 1))
    idx_t = jnp.transpose(inputs, (0, 2, 1))
    out_t = _lookup_t(idx_t, tab_t)
    return jnp.transpose(out_t, (0, 3, 1, 2))


# speedup vs baseline: 7.4270x; 1.0785x over previous
import functools

import jax
import jax.numpy as jnp
from jax import lax
from jax.experimental import pallas as pl
from jax.experimental.pallas import tpu as pltpu
from jax.experimental.pallas import tpu_sc as plsc

N_FIELDS = 26
VOCAB = 100000
EMBED_DIM = 32
BATCH = 1024
HIST = 20
LANES = 16
NVEC = BATCH // LANES
HC = 4
NCHUNK = HIST // HC
NIDX = 2
NST = 4

_mesh = plsc.VectorSubcoreMesh(core_axis_name="c", subcore_axis_name="s")


@functools.partial(
    pl.kernel,
    out_type=jax.ShapeDtypeStruct((N_FIELDS, HIST, EMBED_DIM, BATCH), jnp.float32),
    mesh=_mesh,
    compiler_params=pltpu.CompilerParams(needs_layout_passes=False),
    scratch_types=(
        [pltpu.VMEM((VOCAB,), jnp.float32)]
        + [pltpu.VMEM((HC, BATCH), jnp.int32) for _ in range(NIDX)]
        + [pltpu.VMEM((HC, BATCH), jnp.float32) for _ in range(NST)]
        + [pltpu.SemaphoreType.DMA for _ in range(NIDX + NST)]
    ),
)
def _lookup_t(idx_hbm, tab_hbm, out_hbm, *scr):
    row_v = scr[0]
    idx_bufs = scr[1:1 + NIDX]
    st_bufs = scr[1 + NIDX:1 + NIDX + NST]
    idx_sems = scr[1 + NIDX + NST:1 + 2 * NIDX + NST]
    st_sems = scr[1 + 2 * NIDX + NST:]

    w = lax.axis_index("s") * 2 + lax.axis_index("c")

    def drain(buf, sem):
        if buf.dtype == jnp.int32:
            src = idx_hbm.at[0, pl.ds(0, HC)]
        else:
            src = tab_hbm.at[0, pl.ds(0, HC), pl.ds(0, BATCH)]
        pltpu.make_async_copy(src, buf, sem).wait()

    def gather_chunk(idx_buf, st_buf):
        for hh in range(HC):
            @plsc.parallel_loop(0, NVEC, unroll=8)
            def _body(s):
                sl = pl.ds(s * LANES, LANES)
                st_buf[hh, sl] = plsc.load_gather(row_v, [idx_buf[hh, sl]])

    for j in range(NIDX):
        pltpu.async_copy(idx_hbm.at[0, pl.ds(j * HC, HC)], idx_bufs[j],
                         idx_sems[j])

    def per_field(f, carry):
        pltpu.sync_copy(tab_hbm.at[f, w], row_v)
        for c in range(NCHUNK):
            jb = c % NIDX
            sb = c % NST
            drain(idx_bufs[jb], idx_sems[jb])
            if c < NST:
                @pl.when(f > 0)
                def _():
                    drain(st_bufs[sb], st_sems[sb])
            else:
                drain(st_bufs[sb], st_sems[sb])
            gather_chunk(idx_bufs[jb], st_bufs[sb])
            pltpu.async_copy(st_bufs[sb], out_hbm.at[f, pl.ds(c * HC, HC), w],
                             st_sems[sb])
            nc = c + NIDX
            if nc < NCHUNK:
                pltpu.async_copy(idx_hbm.at[f, pl.ds(nc * HC, HC)],
                                 idx_bufs[jb], idx_sems[jb])
            else:
                nf = jnp.minimum(f + 1, N_FIELDS - 1)
                pltpu.async_copy(
                    idx_hbm.at[nf, pl.ds((nc - NCHUNK) * HC, HC)],
                    idx_bufs[jb], idx_sems[jb])
        return carry

    lax.fori_loop(0, N_FIELDS, per_field, 0)
    for j in range(NIDX):
        drain(idx_bufs[j], idx_sems[j])
    for s in range(NST):
        drain(st_bufs[s], st_sems[s])


def kernel(inputs, tables):
    tab_t = jnp.transpose(tables, (0, 2, 1))
    idx_t = jnp.transpose(inputs, (0, 2, 1))
    out_t = _lookup_t(idx_t, tab_t)
    return jnp.transpose(out_t, (0, 3, 1, 2))
